# Initial kernel scaffold; baseline (speedup 1.0000x reference)
import jax
import jax.numpy as jnp
from jax.experimental import pallas as pl


def kernel(X_converted, mask_percentage, test_geneset, gene_table, mut_table):
    raise NotImplementedError("write your pallas kernel here")


# speedup vs baseline: 2.3757x; 2.3757x over previous
import jax
import jax.numpy as jnp
from jax.experimental import pallas as pl

B = 8
G = 20000
F = 64
GB = 2000
NGB = G // GB


def _body(x_ref, gene_ref, mut_ref, out_ref):
    x = x_ref[0, 0, :]
    oh = (x[:, None] == jax.lax.broadcasted_iota(jnp.int32, (GB, 16), 1))
    mut_emb = jnp.dot(oh.astype(jnp.float32), mut_ref[...],
                      preferred_element_type=jnp.float32)
    out_ref[0] = gene_ref[...] + mut_emb


def kernel(X_converted, mask_percentage, test_geneset, gene_table, mut_table):
    xr = X_converted.astype(jnp.int32).reshape(B * NGB, 1, GB)
    mut_pad = jnp.zeros((16, F), jnp.float32).at[:mut_table.shape[0]].set(mut_table)
    out = pl.pallas_call(
        _body,
        grid=(B * NGB,),
        in_specs=[
            pl.BlockSpec((1, 1, GB), lambda i: (i, 0, 0)),
            pl.BlockSpec((GB, F), lambda i: (i % NGB, 0)),
            pl.BlockSpec((16, F), lambda i: (0, 0)),
        ],
        out_specs=pl.BlockSpec((1, GB, F), lambda i: (i, 0, 0)),
        out_shape=jax.ShapeDtypeStruct((B * NGB, GB, F), jnp.float32),
    )(xr, gene_table, mut_pad)
    return out.reshape(B, G, F)
